# baseline (speedup 1.0000x reference)
import functools

import jax
import jax.numpy as jnp
from jax import lax
from jax.experimental import pallas as pl
from jax.experimental.pallas import tpu as pltpu
from jax.experimental.pallas import tpu_sc as plsc

def _bcast_lane(vec, lane):
    idx = jnp.full((LANES,), lane, jnp.int32)
    return lax.gather(
        vec, idx[:, None],
        lax.GatherDimensionNumbers(
            offset_dims=(), collapsed_slice_dims=(0,), start_index_map=(0,)),
        (1,), mode=lax.GatherScatterMode.PROMISE_IN_BOUNDS)


NC = 2
NS = 16
LANES = 16
CH = 80


def _edge_aggregate(table, gidx, dst, norm, n_nodes, *, scale):
    t_rows, hdim = table.shape
    e_total = gidx.shape[0]
    nw = NC * NS
    ept = e_total // nw
    nchunk = ept // CH
    wpt = 640
    last = n_nodes - (NS - 1) * wpt
    zr = 16
    groups = hdim // LANES
    assert 0 < last <= wpt and last % zr == 0 and wpt % zr == 0

    nbuf = 3
    assert nchunk % nbuf == 2 and nchunk >= 8

    mesh = plsc.VectorSubcoreMesh(core_axis_name="c", subcore_axis_name="s")

    scratch = [
        pltpu.VMEM((ept,), jnp.int32),
        pltpu.VMEM((nbuf, CH), jnp.int32),
        pltpu.VMEM((nbuf, CH, hdim), jnp.float32),
        pltpu.VMEM((zr, hdim), jnp.float32),
        pltpu.VMEM_SHARED((n_nodes, hdim), jnp.float32),
        pltpu.SemaphoreType.DMA,
        pltpu.SemaphoreType.DMA,
        pltpu.SemaphoreType.DMA,
        pltpu.SemaphoreType.DMA,
        pltpu.SemaphoreType.DMA,
        pltpu.SemaphoreType.DMA,
        pltpu.SemaphoreType.DMA,
        pltpu.SemaphoreType.DMA,
        pltpu.SemaphoreType.DMA,
        pltpu.SemaphoreType.DMA,
    ]
    if scale:
        scratch += [
            pltpu.VMEM((nbuf, CH), jnp.float32),
        ]

    def body(table_h, gidx_h, dst_h, norm_h, out_h, idx_v, dstrow_v, rows_v,
             zero_v, agg, sem_i, sem_g0, sem_g1, sem_g2, sem_d0, sem_d1,
             sem_d2, sem_s0, sem_s1, sem_s2, *opt):
        cid = lax.axis_index("c")
        sid = lax.axis_index("s")
        wid = cid * NS + sid
        ebase = pl.multiple_of(wid * ept, 8)
        sem_g = (sem_g0, sem_g1, sem_g2)
        sem_d = (sem_d0, sem_d1, sem_d2)
        sem_s = (sem_s0, sem_s1, sem_s2)
        normrow_v = opt[0] if scale else None

        pltpu.async_copy(gidx_h.at[pl.ds(ebase, ept)], idx_v, sem_i)

        def zfill(i, _):
            row = i // groups
            g = i % groups
            zero_v[row, pl.ds(g * LANES, LANES)] = jnp.zeros((LANES,), jnp.float32)
            return 0
        lax.fori_loop(0, zr * groups, zfill, 0)
        nbase = pl.multiple_of(sid * wpt, 8)

        @pl.when(sid < NS - 1)
        def _zero_full():
            for k in range(wpt // zr):
                pltpu.async_copy(zero_v, agg.at[pl.ds(nbase + k * zr, zr)], sem_s0)
            for k in range(wpt // zr):
                pltpu.make_async_copy(
                    zero_v, agg.at[pl.ds(nbase + k * zr, zr)], sem_s0).wait()

        @pl.when(sid == NS - 1)
        def _zero_last():
            for k in range(last // zr):
                pltpu.async_copy(zero_v, agg.at[pl.ds(nbase + k * zr, zr)], sem_s0)
            for k in range(last // zr):
                pltpu.make_async_copy(
                    zero_v, agg.at[pl.ds(nbase + k * zr, zr)], sem_s0).wait()

        pltpu.make_async_copy(gidx_h.at[pl.ds(ebase, ept)], idx_v, sem_i).wait()

        plsc.subcore_barrier()

        def g_off(j):
            return pl.multiple_of(j * CH, 8)

        def issue_fetch(j, bb):
            off = g_off(j)
            pltpu.async_copy(table_h.at[idx_v.at[pl.ds(off, CH)]],
                             rows_v.at[bb], sem_g[bb])
            pltpu.async_copy(dst_h.at[pl.ds(ebase + off, CH)],
                             dstrow_v.at[bb], sem_d[bb])
            if scale:
                pltpu.async_copy(norm_h.at[pl.ds(ebase + off, CH)],
                                 normrow_v.at[bb], sem_d[bb])

        def wait_fetch(j, bb):
            off = g_off(j)
            pltpu.make_async_copy(table_h.at[idx_v.at[pl.ds(off, CH)]],
                                  rows_v.at[bb], sem_g[bb]).wait()
            pltpu.make_async_copy(dst_h.at[pl.ds(ebase + off, CH)],
                                  dstrow_v.at[bb], sem_d[bb]).wait()
            if scale:
                pltpu.make_async_copy(norm_h.at[pl.ds(ebase + off, CH)],
                                      normrow_v.at[bb], sem_d[bb]).wait()

        def issue_scatter(bb):
            pltpu.async_copy(rows_v.at[bb], agg.at[dstrow_v.at[bb]],
                             sem_s[bb], add=True)

        def wait_scatter(bb):
            pltpu.make_async_copy(rows_v.at[bb], agg.at[dstrow_v.at[bb]],
                                  sem_s[bb]).wait()

        def do_scale(bb):
            if not scale:
                return
            for g16 in range(CH // LANES):
                norm16 = normrow_v[bb, pl.ds(g16 * LANES, LANES)]

                def scale_one(i, _c, g16=g16, norm16=norm16):
                    nb = _bcast_lane(norm16, i)
                    row = g16 * LANES + i
                    for g in range(groups):
                        sl = pl.ds(g * LANES, LANES)
                        rows_v[bb, row, sl] = rows_v[bb, row, sl] * nb
                    return 0
                lax.fori_loop(0, LANES, scale_one, 0)

        def run_chunk(j, q, fetch_next, wait_prev):
            wait_fetch(j, q)
            do_scale(q)
            issue_scatter(q)
            if fetch_next:
                q2 = (q + 2) % nbuf
                if wait_prev:
                    wait_scatter(q2)
                issue_fetch(j + 2, q2)

        issue_fetch(0, 0)
        issue_fetch(1, 1)
        run_chunk(0, 0, True, False)
        run_chunk(1, 1, True, True)
        run_chunk(2, 2, True, True)

        def steady(j3, _):
            j = 3 * j3
            run_chunk(j, 0, True, True)
            run_chunk(j + 1, 1, True, True)
            run_chunk(j + 2, 2, True, True)
            return 0
        lax.fori_loop(1, 1 + (nchunk - 5) // 3, steady, 0)

        run_chunk(nchunk - 2, 0, False, False)
        run_chunk(nchunk - 1, 1, False, False)
        wait_scatter(2)
        wait_scatter(0)
        wait_scatter(1)

        plsc.subcore_barrier()

        @pl.when(sid < NS - 1)
        def _wb_full():
            pltpu.sync_copy(agg.at[pl.ds(nbase, wpt)],
                            out_h.at[cid, pl.ds(nbase, wpt)])

        @pl.when(sid == NS - 1)
        def _wb_last():
            pltpu.sync_copy(agg.at[pl.ds(nbase, last)],
                            out_h.at[cid, pl.ds(nbase, last)])

    f = pl.kernel(
        body,
        out_type=jax.ShapeDtypeStruct((NC, n_nodes, hdim), jnp.float32),
        mesh=mesh,
        scratch_types=scratch,
    )
    return f(table, gidx, dst, norm)


def _tc_rel_gidx(x, w_rel, src2d, etype2d, n_nodes):
    r, d, h = w_rel.shape
    n = x.shape[0]
    bn = 1000
    eb = src2d.shape[1]

    def body(x_ref, w_ref, s_ref, t_ref, o_ref, g_ref):
        for ri in range(r):
            o_ref[ri] = jnp.dot(x_ref[...], w_ref[ri],
                                preferred_element_type=jnp.float32)
        g_ref[...] = t_ref[...] * n_nodes + s_ref[...]

    nbk = src2d.shape[0]
    src3d = src2d.reshape(nbk, 1, eb)
    etype3d = etype2d.reshape(nbk, 1, eb)
    xr, gidx3 = pl.pallas_call(
        body,
        grid=(n // bn,),
        in_specs=[
            pl.BlockSpec((bn, d), lambda i: (i, 0)),
            pl.BlockSpec((r, d, h), lambda i: (0, 0, 0)),
            pl.BlockSpec((1, 1, eb), lambda i: (i, 0, 0)),
            pl.BlockSpec((1, 1, eb), lambda i: (i, 0, 0)),
        ],
        out_specs=[
            pl.BlockSpec((r, bn, h), lambda i: (0, i, 0)),
            pl.BlockSpec((1, 1, eb), lambda i: (i, 0, 0)),
        ],
        out_shape=[
            jax.ShapeDtypeStruct((r, n, h), jnp.float32),
            jax.ShapeDtypeStruct((nbk, 1, eb), jnp.int32),
        ],
    )(x, w_rel, src3d, etype3d)
    return xr, gidx3


def _tc_mid(p, x, w_root, w2, b_rgcn):
    n, d = x.shape
    h = w_root.shape[1]
    bn = 1000

    def body(p_ref, x_ref, wr, wb, b_ref, o1, o2):
        h1 = (p_ref[0] + p_ref[1] + b_ref[...]
              + jnp.dot(x_ref[...], wr[...], preferred_element_type=jnp.float32))
        o1[...] = h1
        o2[...] = jnp.dot(h1, wb[...], preferred_element_type=jnp.float32)

    return pl.pallas_call(
        body,
        grid=(n // bn,),
        in_specs=[
            pl.BlockSpec((2, bn, h), lambda i: (0, i, 0)),
            pl.BlockSpec((bn, d), lambda i: (i, 0)),
            pl.BlockSpec((d, h), lambda i: (0, 0)),
            pl.BlockSpec((h, h), lambda i: (0, 0)),
            pl.BlockSpec((1, h), lambda i: (0, 0)),
        ],
        out_specs=[
            pl.BlockSpec((bn, h), lambda i: (i, 0)),
            pl.BlockSpec((bn, h), lambda i: (i, 0)),
        ],
        out_shape=[
            jax.ShapeDtypeStruct((n, h), jnp.float32),
            jax.ShapeDtypeStruct((n, h), jnp.float32),
        ],
    )(p, x, w_root, w2, b_rgcn)


def _tc_aux(x, h1, wl0, w1, b_lin):
    n, d = x.shape
    h = w1.shape[1]
    bn = 1000

    def body(x_ref, h1_ref, a0, wa, bl, o1, o2):
        o1[...] = jnp.dot(x_ref[...], a0[...],
                          preferred_element_type=jnp.float32) + bl[...]
        o2[...] = jnp.dot(h1_ref[...], wa[...],
                          preferred_element_type=jnp.float32)

    return pl.pallas_call(
        body,
        grid=(n // bn,),
        in_specs=[
            pl.BlockSpec((bn, d), lambda i: (i, 0)),
            pl.BlockSpec((bn, h), lambda i: (i, 0)),
            pl.BlockSpec((d, h), lambda i: (0, 0)),
            pl.BlockSpec((h, h), lambda i: (0, 0)),
            pl.BlockSpec((1, h), lambda i: (0, 0)),
        ],
        out_specs=[
            pl.BlockSpec((bn, h), lambda i: (i, 0)),
            pl.BlockSpec((bn, h), lambda i: (i, 0)),
        ],
        out_shape=[
            jax.ShapeDtypeStruct((n, h), jnp.float32),
            jax.ShapeDtypeStruct((n, h), jnp.float32),
        ],
    )(x, h1, wl0, w1, b_lin)


def _tc_head(xlin, hw1, q, wl1, b_gc, w_smax, b_smax):
    n, h = xlin.shape
    c = w_smax.shape[1]
    bn = 1000

    def body(xl_ref, hw1_ref, q_ref, a1, bg, ws, bs, o_ref):
        h2 = hw1_ref[...] + q_ref[0] + q_ref[1] + bg[...]
        hid = xl_ref[...] + jnp.dot(h2, a1[...],
                                    preferred_element_type=jnp.float32)
        hid = jnp.maximum(hid, 0.0)
        lg = jnp.dot(hid, ws[...], preferred_element_type=jnp.float32) + bs[...]
        m = jnp.max(lg, axis=1, keepdims=True)
        ls = jnp.log(jnp.sum(jnp.exp(lg - m), axis=1, keepdims=True)) + m
        o_ref[...] = lg - ls

    return pl.pallas_call(
        body,
        grid=(n // bn,),
        in_specs=[
            pl.BlockSpec((bn, h), lambda i: (i, 0)),
            pl.BlockSpec((bn, h), lambda i: (i, 0)),
            pl.BlockSpec((2, bn, h), lambda i: (0, i, 0)),
            pl.BlockSpec((h, h), lambda i: (0, 0)),
            pl.BlockSpec((1, h), lambda i: (0, 0)),
            pl.BlockSpec((h, c), lambda i: (0, 0)),
            pl.BlockSpec((1, c), lambda i: (0, 0)),
        ],
        out_specs=pl.BlockSpec((bn, c), lambda i: (i, 0)),
        out_shape=jax.ShapeDtypeStruct((n, c), jnp.float32),
    )(xlin, hw1, q, wl1, b_gc, w_smax, b_smax)


def kernel(x, edge_index, edge_norm, edge_type, W_rel, W_root, b_rgcn,
           W1, W2, b_gc, W_lin, b_lin, W_smax, b_smax):
    n, d = x.shape
    e = edge_index.shape[1]
    r, _, h = W_rel.shape

    src = edge_index[0].astype(jnp.int32)
    dst = edge_index[1].astype(jnp.int32)
    etype = edge_type.astype(jnp.int32)

    nb = 10
    xr2, gidx2 = _tc_rel_gidx(x, W_rel, src.reshape(nb, e // nb),
                              etype.reshape(nb, e // nb), n)
    xr = xr2.reshape(r * n, h)
    gidx = gidx2.reshape(e)
    p1 = _edge_aggregate(xr, gidx, dst, edge_norm, n, scale=True)
    h1, hw2 = _tc_mid(p1, x, W_root, W2, b_rgcn.reshape(1, h))

    p2 = _edge_aggregate(hw2, src, dst, edge_norm, n, scale=False)
    xlin, hw1 = _tc_aux(x, h1, W_lin[:d], W1, b_lin.reshape(1, h))

    return _tc_head(xlin, hw1, p2, W_lin[d:], b_gc.reshape(1, h),
                    W_smax, b_smax.reshape(1, -1))


# speedup vs baseline: 1.0019x; 1.0019x over previous
import functools

import jax
import jax.numpy as jnp
from jax import lax
from jax.experimental import pallas as pl
from jax.experimental.pallas import tpu as pltpu
from jax.experimental.pallas import tpu_sc as plsc

def _bcast_lane(vec, lane):
    idx = jnp.full((LANES,), lane, jnp.int32)
    return lax.gather(
        vec, idx[:, None],
        lax.GatherDimensionNumbers(
            offset_dims=(), collapsed_slice_dims=(0,), start_index_map=(0,)),
        (1,), mode=lax.GatherScatterMode.PROMISE_IN_BOUNDS)


NC = 2
NS = 16
LANES = 16
CH = 80


def _edge_aggregate(table, gidx, dst, norm, n_nodes, *, scale):
    t_rows, hdim = table.shape
    e_total = gidx.shape[0]
    nw = NC * NS
    ept = e_total // nw
    nchunk = ept // CH
    wpt = 640
    last = n_nodes - (NS - 1) * wpt
    zr = 16
    groups = hdim // LANES
    assert 0 < last <= wpt and last % zr == 0 and wpt % zr == 0

    nbuf = 3
    assert nchunk % nbuf == 2 and nchunk >= 8

    mesh = plsc.VectorSubcoreMesh(core_axis_name="c", subcore_axis_name="s")

    scratch = [
        pltpu.VMEM((ept,), jnp.int32),
        pltpu.VMEM((nbuf, CH), jnp.int32),
        pltpu.VMEM((nbuf, CH, hdim), jnp.float32),
        pltpu.VMEM((zr, hdim), jnp.float32),
        pltpu.VMEM_SHARED((n_nodes, hdim), jnp.float32),
        pltpu.SemaphoreType.DMA,
        pltpu.SemaphoreType.DMA,
        pltpu.SemaphoreType.DMA,
        pltpu.SemaphoreType.DMA,
        pltpu.SemaphoreType.DMA,
        pltpu.SemaphoreType.DMA,
        pltpu.SemaphoreType.DMA,
        pltpu.SemaphoreType.DMA,
        pltpu.SemaphoreType.DMA,
        pltpu.SemaphoreType.DMA,
    ]
    if scale:
        scratch += [
            pltpu.VMEM((nbuf, CH), jnp.float32),
        ]

    def body(table_h, gidx_h, dst_h, norm_h, out_h, idx_v, dstrow_v, rows_v,
             zero_v, agg, sem_i, sem_g0, sem_g1, sem_g2, sem_d0, sem_d1,
             sem_d2, sem_s0, sem_s1, sem_s2, *opt):
        cid = lax.axis_index("c")
        sid = lax.axis_index("s")
        wid = cid * NS + sid
        ebase = pl.multiple_of(wid * ept, 8)
        sem_g = (sem_g0, sem_g1, sem_g2)
        sem_d = (sem_d0, sem_d1, sem_d2)
        sem_s = (sem_s0, sem_s1, sem_s2)
        normrow_v = opt[0] if scale else None

        pltpu.async_copy(gidx_h.at[pl.ds(ebase, ept)], idx_v, sem_i)

        def zfill(i, _):
            row = i // groups
            g = i % groups
            zero_v[row, pl.ds(g * LANES, LANES)] = jnp.zeros((LANES,), jnp.float32)
            return 0
        lax.fori_loop(0, zr * groups, zfill, 0)
        nbase = pl.multiple_of(sid * wpt, 8)

        @pl.when(sid < NS - 1)
        def _zero_full():
            for k in range(wpt // zr):
                pltpu.async_copy(zero_v, agg.at[pl.ds(nbase + k * zr, zr)], sem_s0)
            for k in range(wpt // zr):
                pltpu.make_async_copy(
                    zero_v, agg.at[pl.ds(nbase + k * zr, zr)], sem_s0).wait()

        @pl.when(sid == NS - 1)
        def _zero_last():
            for k in range(last // zr):
                pltpu.async_copy(zero_v, agg.at[pl.ds(nbase + k * zr, zr)], sem_s0)
            for k in range(last // zr):
                pltpu.make_async_copy(
                    zero_v, agg.at[pl.ds(nbase + k * zr, zr)], sem_s0).wait()

        pltpu.make_async_copy(gidx_h.at[pl.ds(ebase, ept)], idx_v, sem_i).wait()

        plsc.subcore_barrier()

        def g_off(j):
            return pl.multiple_of(j * CH, 8)

        def issue_fetch(j, bb):
            off = g_off(j)
            pltpu.async_copy(table_h.at[idx_v.at[pl.ds(off, CH)]],
                             rows_v.at[bb], sem_g[bb])
            pltpu.async_copy(dst_h.at[pl.ds(ebase + off, CH)],
                             dstrow_v.at[bb], sem_d[bb])
            if scale:
                pltpu.async_copy(norm_h.at[pl.ds(ebase + off, CH)],
                                 normrow_v.at[bb], sem_d[bb])

        def wait_fetch(j, bb):
            off = g_off(j)
            pltpu.make_async_copy(table_h.at[idx_v.at[pl.ds(off, CH)]],
                                  rows_v.at[bb], sem_g[bb]).wait()
            pltpu.make_async_copy(dst_h.at[pl.ds(ebase + off, CH)],
                                  dstrow_v.at[bb], sem_d[bb]).wait()
            if scale:
                pltpu.make_async_copy(norm_h.at[pl.ds(ebase + off, CH)],
                                      normrow_v.at[bb], sem_d[bb]).wait()

        def issue_scatter(bb):
            pltpu.async_copy(rows_v.at[bb], agg.at[dstrow_v.at[bb]],
                             sem_s[bb], add=True)

        def wait_scatter(bb):
            pltpu.make_async_copy(rows_v.at[bb], agg.at[dstrow_v.at[bb]],
                                  sem_s[bb]).wait()

        def do_scale(bb):
            if not scale:
                return
            for g16 in range(CH // LANES):
                norm16 = normrow_v[bb, pl.ds(g16 * LANES, LANES)]

                def scale_one(i, _c, g16=g16, norm16=norm16):
                    nb = _bcast_lane(norm16, i)
                    row = g16 * LANES + i
                    for g in range(groups):
                        sl = pl.ds(g * LANES, LANES)
                        rows_v[bb, row, sl] = rows_v[bb, row, sl] * nb
                    return 0
                lax.fori_loop(0, LANES, scale_one, 0)

        def run_chunk(j, q, fetch_next, wait_prev):
            wait_fetch(j, q)
            do_scale(q)
            issue_scatter(q)
            if fetch_next:
                q2 = (q + 2) % nbuf
                if wait_prev:
                    wait_scatter(q2)
                issue_fetch(j + 2, q2)

        issue_fetch(0, 0)
        issue_fetch(1, 1)
        run_chunk(0, 0, True, False)
        run_chunk(1, 1, True, True)
        run_chunk(2, 2, True, True)

        def steady(j3, _):
            j = 3 * j3
            run_chunk(j, 0, True, True)
            run_chunk(j + 1, 1, True, True)
            run_chunk(j + 2, 2, True, True)
            return 0
        lax.fori_loop(1, 1 + (nchunk - 5) // 3, steady, 0)

        run_chunk(nchunk - 2, 0, False, False)
        run_chunk(nchunk - 1, 1, False, False)
        wait_scatter(2)
        wait_scatter(0)
        wait_scatter(1)

        plsc.subcore_barrier()

        @pl.when(sid < NS - 1)
        def _wb_full():
            pltpu.sync_copy(agg.at[pl.ds(nbase, wpt)],
                            out_h.at[cid, pl.ds(nbase, wpt)])

        @pl.when(sid == NS - 1)
        def _wb_last():
            pltpu.sync_copy(agg.at[pl.ds(nbase, last)],
                            out_h.at[cid, pl.ds(nbase, last)])

    f = pl.kernel(
        body,
        out_type=jax.ShapeDtypeStruct((NC, n_nodes, hdim), jnp.float32),
        mesh=mesh,
        scratch_types=scratch,
    )
    return f(table, gidx, dst, norm)


def _tc_rel_gidx(x, w_rel, src2d, etype2d, n_nodes):
    r, d, h = w_rel.shape
    n = x.shape[0]
    bn = 1000
    eb = src2d.shape[1]

    def body(x_ref, w_ref, s_ref, t_ref, o_ref, g_ref):
        for ri in range(r):
            o_ref[ri] = jnp.dot(x_ref[...], w_ref[ri],
                                preferred_element_type=jnp.float32)
        g_ref[...] = t_ref[...] * n_nodes + s_ref[...]

    nbk = src2d.shape[0]
    src3d = src2d.reshape(nbk, 1, eb)
    etype3d = etype2d.reshape(nbk, 1, eb)
    xr, gidx3 = pl.pallas_call(
        body,
        grid=(n // bn,),
        in_specs=[
            pl.BlockSpec((bn, d), lambda i: (i, 0)),
            pl.BlockSpec((r, d, h), lambda i: (0, 0, 0)),
            pl.BlockSpec((1, 1, eb), lambda i: (i, 0, 0)),
            pl.BlockSpec((1, 1, eb), lambda i: (i, 0, 0)),
        ],
        out_specs=[
            pl.BlockSpec((r, bn, h), lambda i: (0, i, 0)),
            pl.BlockSpec((1, 1, eb), lambda i: (i, 0, 0)),
        ],
        out_shape=[
            jax.ShapeDtypeStruct((r, n, h), jnp.float32),
            jax.ShapeDtypeStruct((nbk, 1, eb), jnp.int32),
        ],
    )(x, w_rel, src3d, etype3d)
    return xr, gidx3


def _tc_mid(p, x, w_root, w2, b_rgcn):
    n, d = x.shape
    h = w_root.shape[1]
    bn = 1000

    def body(p_ref, x_ref, wr, wb, b_ref, o1, o2):
        h1 = (p_ref[0] + p_ref[1] + b_ref[...]
              + jnp.dot(x_ref[...], wr[...], preferred_element_type=jnp.float32))
        o1[...] = h1
        o2[...] = jnp.dot(h1, wb[...], preferred_element_type=jnp.float32)

    return pl.pallas_call(
        body,
        grid=(n // bn,),
        in_specs=[
            pl.BlockSpec((2, bn, h), lambda i: (0, i, 0)),
            pl.BlockSpec((bn, d), lambda i: (i, 0)),
            pl.BlockSpec((d, h), lambda i: (0, 0)),
            pl.BlockSpec((h, h), lambda i: (0, 0)),
            pl.BlockSpec((1, h), lambda i: (0, 0)),
        ],
        out_specs=[
            pl.BlockSpec((bn, h), lambda i: (i, 0)),
            pl.BlockSpec((bn, h), lambda i: (i, 0)),
        ],
        out_shape=[
            jax.ShapeDtypeStruct((n, h), jnp.float32),
            jax.ShapeDtypeStruct((n, h), jnp.float32),
        ],
    )(p, x, w_root, w2, b_rgcn)


def _tc_head(x, h1, q, wl0, wl1, w1, b_lin, b_gc, w_smax, b_smax):
    n, d = x.shape
    h = w1.shape[1]
    c = w_smax.shape[1]
    bn = 1000

    def body(x_ref, h1_ref, q_ref, a0, a1, wa, bl, bg, ws, bs, o_ref):
        hw1 = jnp.dot(h1_ref[...], wa[...], preferred_element_type=jnp.float32)
        h2 = hw1 + q_ref[0] + q_ref[1] + bg[...]
        hid = jnp.dot(x_ref[...], a0[...], preferred_element_type=jnp.float32)
        hid = hid + jnp.dot(h2, a1[...], preferred_element_type=jnp.float32)
        hid = jnp.maximum(hid + bl[...], 0.0)
        lg = jnp.dot(hid, ws[...], preferred_element_type=jnp.float32) + bs[...]
        m = jnp.max(lg, axis=1, keepdims=True)
        ls = jnp.log(jnp.sum(jnp.exp(lg - m), axis=1, keepdims=True)) + m
        o_ref[...] = lg - ls

    return pl.pallas_call(
        body,
        grid=(n // bn,),
        in_specs=[
            pl.BlockSpec((bn, d), lambda i: (i, 0)),
            pl.BlockSpec((bn, h), lambda i: (i, 0)),
            pl.BlockSpec((2, bn, h), lambda i: (0, i, 0)),
            pl.BlockSpec((d, h), lambda i: (0, 0)),
            pl.BlockSpec((h, h), lambda i: (0, 0)),
            pl.BlockSpec((h, h), lambda i: (0, 0)),
            pl.BlockSpec((1, h), lambda i: (0, 0)),
            pl.BlockSpec((1, h), lambda i: (0, 0)),
            pl.BlockSpec((h, c), lambda i: (0, 0)),
            pl.BlockSpec((1, c), lambda i: (0, 0)),
        ],
        out_specs=pl.BlockSpec((bn, c), lambda i: (i, 0)),
        out_shape=jax.ShapeDtypeStruct((n, c), jnp.float32),
    )(x, h1, q, wl0, wl1, w1, b_lin, b_gc, w_smax, b_smax)


def kernel(x, edge_index, edge_norm, edge_type, W_rel, W_root, b_rgcn,
           W1, W2, b_gc, W_lin, b_lin, W_smax, b_smax):
    n, d = x.shape
    e = edge_index.shape[1]
    r, _, h = W_rel.shape

    src = edge_index[0].astype(jnp.int32)
    dst = edge_index[1].astype(jnp.int32)
    etype = edge_type.astype(jnp.int32)

    nb = 10
    xr2, gidx2 = _tc_rel_gidx(x, W_rel, src.reshape(nb, e // nb),
                              etype.reshape(nb, e // nb), n)
    xr = xr2.reshape(r * n, h)
    gidx = gidx2.reshape(e)
    p1 = _edge_aggregate(xr, gidx, dst, edge_norm, n, scale=True)
    h1, hw2 = _tc_mid(p1, x, W_root, W2, b_rgcn.reshape(1, h))

    p2 = _edge_aggregate(hw2, src, dst, edge_norm, n, scale=False)

    return _tc_head(x, h1, p2, W_lin[:d], W_lin[d:], W1, b_lin.reshape(1, h),
                    b_gc.reshape(1, h), W_smax, b_smax.reshape(1, -1))


# speedup vs baseline: 1.0021x; 1.0003x over previous
import functools

import jax
import jax.numpy as jnp
from jax import lax
from jax.experimental import pallas as pl
from jax.experimental.pallas import tpu as pltpu
from jax.experimental.pallas import tpu_sc as plsc

def _bcast_lane(vec, lane):
    idx = jnp.full((LANES,), lane, jnp.int32)
    return lax.gather(
        vec, idx[:, None],
        lax.GatherDimensionNumbers(
            offset_dims=(), collapsed_slice_dims=(0,), start_index_map=(0,)),
        (1,), mode=lax.GatherScatterMode.PROMISE_IN_BOUNDS)


NC = 2
NS = 16
LANES = 16
CH = 80


def _edge_aggregate(table, gidx, dst, norm, n_nodes, *, scale):
    t_rows, hdim = table.shape
    e_total = gidx.shape[0]
    nw = NC * NS
    ept = e_total // nw
    nchunk = ept // CH
    wpt = 640
    last = n_nodes - (NS - 1) * wpt
    zr = 16
    groups = hdim // LANES
    assert 0 < last <= wpt and last % zr == 0 and wpt % zr == 0

    nbuf = 3
    assert nchunk % nbuf == 2 and nchunk >= 8

    mesh = plsc.VectorSubcoreMesh(core_axis_name="c", subcore_axis_name="s")

    scratch = [
        pltpu.VMEM((ept,), jnp.int32),
        pltpu.VMEM((nbuf, CH), jnp.int32),
        pltpu.VMEM((nbuf, CH, hdim), jnp.float32),
        pltpu.VMEM((zr, hdim), jnp.float32),
        pltpu.VMEM_SHARED((n_nodes, hdim), jnp.float32),
        pltpu.SemaphoreType.DMA,
        pltpu.SemaphoreType.DMA,
        pltpu.SemaphoreType.DMA,
        pltpu.SemaphoreType.DMA,
        pltpu.SemaphoreType.DMA,
        pltpu.SemaphoreType.DMA,
        pltpu.SemaphoreType.DMA,
        pltpu.SemaphoreType.DMA,
        pltpu.SemaphoreType.DMA,
        pltpu.SemaphoreType.DMA,
    ]
    if scale:
        scratch += [
            pltpu.VMEM((nbuf, CH), jnp.float32),
        ]

    def body(table_h, gidx_h, dst_h, norm_h, out_h, idx_v, dstrow_v, rows_v,
             zero_v, agg, sem_i, sem_g0, sem_g1, sem_g2, sem_d0, sem_d1,
             sem_d2, sem_s0, sem_s1, sem_s2, *opt):
        cid = lax.axis_index("c")
        sid = lax.axis_index("s")
        wid = cid * NS + sid
        ebase = pl.multiple_of(wid * ept, 8)
        sem_g = (sem_g0, sem_g1, sem_g2)
        sem_d = (sem_d0, sem_d1, sem_d2)
        sem_s = (sem_s0, sem_s1, sem_s2)
        normrow_v = opt[0] if scale else None

        pltpu.async_copy(gidx_h.at[pl.ds(ebase, ept)], idx_v, sem_i)

        def zfill(i, _):
            row = i // groups
            g = i % groups
            zero_v[row, pl.ds(g * LANES, LANES)] = jnp.zeros((LANES,), jnp.float32)
            return 0
        lax.fori_loop(0, zr * groups, zfill, 0)
        nbase = pl.multiple_of(sid * wpt, 8)

        @pl.when(sid < NS - 1)
        def _zero_full():
            for k in range(wpt // zr):
                pltpu.async_copy(zero_v, agg.at[pl.ds(nbase + k * zr, zr)], sem_s0)
            for k in range(wpt // zr):
                pltpu.make_async_copy(
                    zero_v, agg.at[pl.ds(nbase + k * zr, zr)], sem_s0).wait()

        @pl.when(sid == NS - 1)
        def _zero_last():
            for k in range(last // zr):
                pltpu.async_copy(zero_v, agg.at[pl.ds(nbase + k * zr, zr)], sem_s0)
            for k in range(last // zr):
                pltpu.make_async_copy(
                    zero_v, agg.at[pl.ds(nbase + k * zr, zr)], sem_s0).wait()

        pltpu.make_async_copy(gidx_h.at[pl.ds(ebase, ept)], idx_v, sem_i).wait()

        plsc.subcore_barrier()

        def g_off(j):
            return pl.multiple_of(j * CH, 8)

        def issue_fetch(j, bb):
            off = g_off(j)
            pltpu.async_copy(table_h.at[idx_v.at[pl.ds(off, CH)]],
                             rows_v.at[bb], sem_g[bb])
            pltpu.async_copy(dst_h.at[pl.ds(ebase + off, CH)],
                             dstrow_v.at[bb], sem_d[bb])
            if scale:
                pltpu.async_copy(norm_h.at[pl.ds(ebase + off, CH)],
                                 normrow_v.at[bb], sem_d[bb])

        def wait_fetch(j, bb):
            off = g_off(j)
            pltpu.make_async_copy(table_h.at[idx_v.at[pl.ds(off, CH)]],
                                  rows_v.at[bb], sem_g[bb]).wait()
            pltpu.make_async_copy(dst_h.at[pl.ds(ebase + off, CH)],
                                  dstrow_v.at[bb], sem_d[bb]).wait()
            if scale:
                pltpu.make_async_copy(norm_h.at[pl.ds(ebase + off, CH)],
                                      normrow_v.at[bb], sem_d[bb]).wait()

        def issue_scatter(bb):
            pltpu.async_copy(rows_v.at[bb], agg.at[dstrow_v.at[bb]],
                             sem_s[bb], add=True)

        def wait_scatter(bb):
            pltpu.make_async_copy(rows_v.at[bb], agg.at[dstrow_v.at[bb]],
                                  sem_s[bb]).wait()

        def do_scale(bb):
            if not scale:
                return
            for g16 in range(CH // LANES):
                norm16 = normrow_v[bb, pl.ds(g16 * LANES, LANES)]

                def scale_one(i, _c, g16=g16, norm16=norm16):
                    nb = _bcast_lane(norm16, i)
                    row = g16 * LANES + i
                    for g in range(groups):
                        sl = pl.ds(g * LANES, LANES)
                        rows_v[bb, row, sl] = rows_v[bb, row, sl] * nb
                    return 0
                lax.fori_loop(0, LANES, scale_one, 0)

        def run_chunk(j, q, fetch_next, wait_prev):
            wait_fetch(j, q)
            do_scale(q)
            issue_scatter(q)
            if fetch_next:
                q2 = (q + 2) % nbuf
                if wait_prev:
                    wait_scatter(q2)
                issue_fetch(j + 2, q2)

        issue_fetch(0, 0)
        issue_fetch(1, 1)
        run_chunk(0, 0, True, False)
        run_chunk(1, 1, True, True)
        run_chunk(2, 2, True, True)

        def steady(j3, _):
            j = 3 * j3
            run_chunk(j, 0, True, True)
            run_chunk(j + 1, 1, True, True)
            run_chunk(j + 2, 2, True, True)
            return 0
        lax.fori_loop(1, 1 + (nchunk - 5) // 3, steady, 0)

        run_chunk(nchunk - 2, 0, False, False)
        run_chunk(nchunk - 1, 1, False, False)
        wait_scatter(2)
        wait_scatter(0)
        wait_scatter(1)

        plsc.subcore_barrier()

        @pl.when(sid < NS - 1)
        def _wb_full():
            pltpu.sync_copy(agg.at[pl.ds(nbase, wpt)],
                            out_h.at[cid, pl.ds(nbase, wpt)])

        @pl.when(sid == NS - 1)
        def _wb_last():
            pltpu.sync_copy(agg.at[pl.ds(nbase, last)],
                            out_h.at[cid, pl.ds(nbase, last)])

    f = pl.kernel(
        body,
        out_type=jax.ShapeDtypeStruct((NC, n_nodes, hdim), jnp.float32),
        mesh=mesh,
        scratch_types=scratch,
    )
    return f(table, gidx, dst, norm)


def _tc_rel_gidx(x, w_rel, src2d, etype2d, n_nodes):
    r, d, h = w_rel.shape
    n = x.shape[0]
    bn = 1000
    eb = src2d.shape[1]

    def body(x_ref, w_ref, s_ref, t_ref, o_ref, g_ref):
        xb = x_ref[...].astype(jnp.bfloat16)
        for ri in range(r):
            o_ref[ri] = jnp.dot(xb, w_ref[ri].astype(jnp.bfloat16),
                                preferred_element_type=jnp.float32)
        g_ref[...] = t_ref[...] * n_nodes + s_ref[...]

    nbk = src2d.shape[0]
    src3d = src2d.reshape(nbk, 1, eb)
    etype3d = etype2d.reshape(nbk, 1, eb)
    xr, gidx3 = pl.pallas_call(
        body,
        grid=(n // bn,),
        in_specs=[
            pl.BlockSpec((bn, d), lambda i: (i, 0)),
            pl.BlockSpec((r, d, h), lambda i: (0, 0, 0)),
            pl.BlockSpec((1, 1, eb), lambda i: (i, 0, 0)),
            pl.BlockSpec((1, 1, eb), lambda i: (i, 0, 0)),
        ],
        out_specs=[
            pl.BlockSpec((r, bn, h), lambda i: (0, i, 0)),
            pl.BlockSpec((1, 1, eb), lambda i: (i, 0, 0)),
        ],
        out_shape=[
            jax.ShapeDtypeStruct((r, n, h), jnp.float32),
            jax.ShapeDtypeStruct((nbk, 1, eb), jnp.int32),
        ],
    )(x, w_rel, src3d, etype3d)
    return xr, gidx3


def _tc_mid(p, x, w_root, w2, b_rgcn):
    n, d = x.shape
    h = w_root.shape[1]
    bn = 1000

    def body(p_ref, x_ref, wr, wb, b_ref, o1, o2):
        h1 = (p_ref[0] + p_ref[1] + b_ref[...]
              + jnp.dot(x_ref[...], wr[...], preferred_element_type=jnp.float32))
        o1[...] = h1
        o2[...] = jnp.dot(h1, wb[...], preferred_element_type=jnp.float32)

    return pl.pallas_call(
        body,
        grid=(n // bn,),
        in_specs=[
            pl.BlockSpec((2, bn, h), lambda i: (0, i, 0)),
            pl.BlockSpec((bn, d), lambda i: (i, 0)),
            pl.BlockSpec((d, h), lambda i: (0, 0)),
            pl.BlockSpec((h, h), lambda i: (0, 0)),
            pl.BlockSpec((1, h), lambda i: (0, 0)),
        ],
        out_specs=[
            pl.BlockSpec((bn, h), lambda i: (i, 0)),
            pl.BlockSpec((bn, h), lambda i: (i, 0)),
        ],
        out_shape=[
            jax.ShapeDtypeStruct((n, h), jnp.float32),
            jax.ShapeDtypeStruct((n, h), jnp.float32),
        ],
    )(p, x, w_root, w2, b_rgcn)


def _tc_head(x, h1, q, wl0, wl1, w1, b_lin, b_gc, w_smax, b_smax):
    n, d = x.shape
    h = w1.shape[1]
    c = w_smax.shape[1]
    bn = 1000

    def body(x_ref, h1_ref, q_ref, a0, a1, wa, bl, bg, ws, bs, o_ref):
        hw1 = jnp.dot(h1_ref[...], wa[...], preferred_element_type=jnp.float32)
        h2 = hw1 + q_ref[0] + q_ref[1] + bg[...]
        hid = jnp.dot(x_ref[...], a0[...], preferred_element_type=jnp.float32)
        hid = hid + jnp.dot(h2, a1[...], preferred_element_type=jnp.float32)
        hid = jnp.maximum(hid + bl[...], 0.0)
        lg = jnp.dot(hid, ws[...], preferred_element_type=jnp.float32) + bs[...]
        m = jnp.max(lg, axis=1, keepdims=True)
        ls = jnp.log(jnp.sum(jnp.exp(lg - m), axis=1, keepdims=True)) + m
        o_ref[...] = lg - ls

    return pl.pallas_call(
        body,
        grid=(n // bn,),
        in_specs=[
            pl.BlockSpec((bn, d), lambda i: (i, 0)),
            pl.BlockSpec((bn, h), lambda i: (i, 0)),
            pl.BlockSpec((2, bn, h), lambda i: (0, i, 0)),
            pl.BlockSpec((d, h), lambda i: (0, 0)),
            pl.BlockSpec((h, h), lambda i: (0, 0)),
            pl.BlockSpec((h, h), lambda i: (0, 0)),
            pl.BlockSpec((1, h), lambda i: (0, 0)),
            pl.BlockSpec((1, h), lambda i: (0, 0)),
            pl.BlockSpec((h, c), lambda i: (0, 0)),
            pl.BlockSpec((1, c), lambda i: (0, 0)),
        ],
        out_specs=pl.BlockSpec((bn, c), lambda i: (i, 0)),
        out_shape=jax.ShapeDtypeStruct((n, c), jnp.float32),
    )(x, h1, q, wl0, wl1, w1, b_lin, b_gc, w_smax, b_smax)


def kernel(x, edge_index, edge_norm, edge_type, W_rel, W_root, b_rgcn,
           W1, W2, b_gc, W_lin, b_lin, W_smax, b_smax):
    n, d = x.shape
    e = edge_index.shape[1]
    r, _, h = W_rel.shape

    src = edge_index[0].astype(jnp.int32)
    dst = edge_index[1].astype(jnp.int32)
    etype = edge_type.astype(jnp.int32)

    nb = 10
    xr2, gidx2 = _tc_rel_gidx(x, W_rel, src.reshape(nb, e // nb),
                              etype.reshape(nb, e // nb), n)
    xr = xr2.reshape(r * n, h)
    gidx = gidx2.reshape(e)
    p1 = _edge_aggregate(xr, gidx, dst, edge_norm, n, scale=True)
    h1, hw2 = _tc_mid(p1, x, W_root, W2, b_rgcn.reshape(1, h))

    p2 = _edge_aggregate(hw2, src, dst, edge_norm, n, scale=False)

    return _tc_head(x, h1, p2, W_lin[:d], W_lin[d:], W1, b_lin.reshape(1, h),
                    b_gc.reshape(1, h), W_smax, b_smax.reshape(1, -1))


# speedup vs baseline: 1.0291x; 1.0270x over previous
import functools

import jax
import jax.numpy as jnp
from jax import lax
from jax.experimental import pallas as pl
from jax.experimental.pallas import tpu as pltpu
from jax.experimental.pallas import tpu_sc as plsc

def _bcast_lane(vec, lane):
    idx = jnp.full((LANES,), lane, jnp.int32)
    return lax.gather(
        vec, idx[:, None],
        lax.GatherDimensionNumbers(
            offset_dims=(), collapsed_slice_dims=(0,), start_index_map=(0,)),
        (1,), mode=lax.GatherScatterMode.PROMISE_IN_BOUNDS)


NC = 2
NS = 16
LANES = 16
CH = 80


def _edge_aggregate(table, gidx, dst, norm, n_nodes, *, scale):
    t_rows, hdim = table.shape
    e_total = gidx.shape[0]
    nw = NC * NS
    ept = e_total // nw
    nchunk = ept // CH
    wpt = 640
    last = n_nodes - (NS - 1) * wpt
    zr = 16
    groups = hdim // LANES
    assert 0 < last <= wpt and last % zr == 0 and wpt % zr == 0

    nbuf = 3
    assert nchunk % nbuf == 2 and nchunk >= 8

    mesh = plsc.VectorSubcoreMesh(core_axis_name="c", subcore_axis_name="s")

    scratch = [
        pltpu.VMEM((ept,), jnp.int32),
        pltpu.VMEM((nbuf, CH), jnp.int32),
        pltpu.VMEM((nbuf, CH, hdim), jnp.float32),
        pltpu.VMEM((zr, hdim), jnp.float32),
        pltpu.VMEM_SHARED((n_nodes, hdim), jnp.float32),
        pltpu.SemaphoreType.DMA,
        pltpu.SemaphoreType.DMA,
        pltpu.SemaphoreType.DMA,
        pltpu.SemaphoreType.DMA,
        pltpu.SemaphoreType.DMA,
        pltpu.SemaphoreType.DMA,
        pltpu.SemaphoreType.DMA,
        pltpu.SemaphoreType.DMA,
        pltpu.SemaphoreType.DMA,
        pltpu.SemaphoreType.DMA,
    ]
    if scale:
        scratch += [
            pltpu.VMEM((nbuf, CH), jnp.float32),
        ]

    def body(table_h, gidx_h, dst_h, norm_h, out_h, idx_v, dstrow_v, rows_v,
             zero_v, agg, sem_i, sem_g0, sem_g1, sem_g2, sem_d0, sem_d1,
             sem_d2, sem_s0, sem_s1, sem_s2, *opt):
        cid = lax.axis_index("c")
        sid = lax.axis_index("s")
        wid = cid * NS + sid
        ebase = pl.multiple_of(wid * ept, 8)
        sem_g = (sem_g0, sem_g1, sem_g2)
        sem_d = (sem_d0, sem_d1, sem_d2)
        sem_s = (sem_s0, sem_s1, sem_s2)
        normrow_v = opt[0] if scale else None

        pltpu.async_copy(gidx_h.at[pl.ds(ebase, ept)], idx_v, sem_i)

        def zfill(i, _):
            row = i // groups
            g = i % groups
            zero_v[row, pl.ds(g * LANES, LANES)] = jnp.zeros((LANES,), jnp.float32)
            return 0
        lax.fori_loop(0, zr * groups, zfill, 0)
        nbase = pl.multiple_of(sid * wpt, 8)

        @pl.when(sid < NS - 1)
        def _zero_full():
            for k in range(wpt // zr):
                pltpu.async_copy(zero_v, agg.at[pl.ds(nbase + k * zr, zr)], sem_s0)
            for k in range(wpt // zr):
                pltpu.make_async_copy(
                    zero_v, agg.at[pl.ds(nbase + k * zr, zr)], sem_s0).wait()

        @pl.when(sid == NS - 1)
        def _zero_last():
            for k in range(last // zr):
                pltpu.async_copy(zero_v, agg.at[pl.ds(nbase + k * zr, zr)], sem_s0)
            for k in range(last // zr):
                pltpu.make_async_copy(
                    zero_v, agg.at[pl.ds(nbase + k * zr, zr)], sem_s0).wait()

        pltpu.make_async_copy(gidx_h.at[pl.ds(ebase, ept)], idx_v, sem_i).wait()

        plsc.subcore_barrier()

        def g_off(j):
            return pl.multiple_of(j * CH, 8)

        def issue_fetch(j, bb):
            off = g_off(j)
            pltpu.async_copy(table_h.at[idx_v.at[pl.ds(off, CH)]],
                             rows_v.at[bb], sem_g[bb])
            pltpu.async_copy(dst_h.at[pl.ds(ebase + off, CH)],
                             dstrow_v.at[bb], sem_d[bb])
            if scale:
                pltpu.async_copy(norm_h.at[pl.ds(ebase + off, CH)],
                                 normrow_v.at[bb], sem_d[bb])

        def wait_fetch(j, bb):
            off = g_off(j)
            pltpu.make_async_copy(table_h.at[idx_v.at[pl.ds(off, CH)]],
                                  rows_v.at[bb], sem_g[bb]).wait()
            pltpu.make_async_copy(dst_h.at[pl.ds(ebase + off, CH)],
                                  dstrow_v.at[bb], sem_d[bb]).wait()
            if scale:
                pltpu.make_async_copy(norm_h.at[pl.ds(ebase + off, CH)],
                                      normrow_v.at[bb], sem_d[bb]).wait()

        def issue_scatter(bb):
            pltpu.async_copy(rows_v.at[bb], agg.at[dstrow_v.at[bb]],
                             sem_s[bb], add=True)

        def wait_scatter(bb):
            pltpu.make_async_copy(rows_v.at[bb], agg.at[dstrow_v.at[bb]],
                                  sem_s[bb]).wait()

        def do_scale(bb):
            if not scale:
                return
            for g16 in range(CH // LANES):
                norm16 = normrow_v[bb, pl.ds(g16 * LANES, LANES)]

                def scale_one(i, _c, g16=g16, norm16=norm16):
                    nb = _bcast_lane(norm16, i)
                    row = g16 * LANES + i
                    for g in range(groups):
                        sl = pl.ds(g * LANES, LANES)
                        rows_v[bb, row, sl] = rows_v[bb, row, sl] * nb
                    return 0
                lax.fori_loop(0, LANES, scale_one, 0)

        def run_chunk(j, q, fetch_next, wait_prev):
            wait_fetch(j, q)
            do_scale(q)
            issue_scatter(q)
            if fetch_next:
                q2 = (q + 2) % nbuf
                if wait_prev:
                    wait_scatter(q2)
                issue_fetch(j + 2, q2)

        issue_fetch(0, 0)
        issue_fetch(1, 1)
        run_chunk(0, 0, True, False)
        run_chunk(1, 1, True, True)
        run_chunk(2, 2, True, True)

        def steady(j3, _):
            j = 3 * j3
            run_chunk(j, 0, True, True)
            run_chunk(j + 1, 1, True, True)
            run_chunk(j + 2, 2, True, True)
            return 0
        lax.fori_loop(1, 1 + (nchunk - 5) // 3, steady, 0)

        run_chunk(nchunk - 2, 0, False, False)
        run_chunk(nchunk - 1, 1, False, False)
        wait_scatter(2)
        wait_scatter(0)
        wait_scatter(1)

        plsc.subcore_barrier()

        @pl.when(sid < NS - 1)
        def _wb_full():
            pltpu.sync_copy(agg.at[pl.ds(nbase, wpt)],
                            out_h.at[cid, pl.ds(nbase, wpt)])

        @pl.when(sid == NS - 1)
        def _wb_last():
            pltpu.sync_copy(agg.at[pl.ds(nbase, last)],
                            out_h.at[cid, pl.ds(nbase, last)])

    f = pl.kernel(
        body,
        out_type=jax.ShapeDtypeStruct((NC, n_nodes, hdim), jnp.float32),
        mesh=mesh,
        scratch_types=scratch,
    )
    return f(table, gidx, dst, norm)


def _tc_rel(x, w_rel):
    r, d, h = w_rel.shape
    n = x.shape[0]
    bn = 1000

    def body(x_ref, w_ref, o_ref):
        xb = x_ref[...].astype(jnp.bfloat16)
        for ri in range(r):
            o_ref[ri] = jnp.dot(xb, w_ref[ri].astype(jnp.bfloat16),
                                preferred_element_type=jnp.float32)

    return pl.pallas_call(
        body,
        grid=(n // bn,),
        in_specs=[
            pl.BlockSpec((bn, d), lambda i: (i, 0)),
            pl.BlockSpec((r, d, h), lambda i: (0, 0, 0)),
        ],
        out_specs=pl.BlockSpec((r, bn, h), lambda i: (0, i, 0)),
        out_shape=jax.ShapeDtypeStruct((r, n, h), jnp.float32),
    )(x, w_rel)


def _tc_gidx(src, etype, n_nodes):
    e = src.shape[0]

    def body(s_ref, t_ref, g_ref):
        g_ref[...] = t_ref[...] * n_nodes + s_ref[...]

    return pl.pallas_call(
        body,
        out_shape=jax.ShapeDtypeStruct((e,), jnp.int32),
    )(src, etype)


def _tc_mid(p, x, w_root, w2, b_rgcn):
    n, d = x.shape
    h = w_root.shape[1]
    bn = 1000

    def body(p_ref, x_ref, wr, wb, b_ref, o1, o2):
        h1 = (p_ref[0] + p_ref[1] + b_ref[...]
              + jnp.dot(x_ref[...], wr[...], preferred_element_type=jnp.float32))
        o1[...] = h1
        o2[...] = jnp.dot(h1, wb[...], preferred_element_type=jnp.float32)

    return pl.pallas_call(
        body,
        grid=(n // bn,),
        in_specs=[
            pl.BlockSpec((2, bn, h), lambda i: (0, i, 0)),
            pl.BlockSpec((bn, d), lambda i: (i, 0)),
            pl.BlockSpec((d, h), lambda i: (0, 0)),
            pl.BlockSpec((h, h), lambda i: (0, 0)),
            pl.BlockSpec((1, h), lambda i: (0, 0)),
        ],
        out_specs=[
            pl.BlockSpec((bn, h), lambda i: (i, 0)),
            pl.BlockSpec((bn, h), lambda i: (i, 0)),
        ],
        out_shape=[
            jax.ShapeDtypeStruct((n, h), jnp.float32),
            jax.ShapeDtypeStruct((n, h), jnp.float32),
        ],
    )(p, x, w_root, w2, b_rgcn)


def _tc_head(x, h1, q, wl0, wl1, w1, b_lin, b_gc, w_smax, b_smax):
    n, d = x.shape
    h = w1.shape[1]
    c = w_smax.shape[1]
    bn = 1000

    def body(x_ref, h1_ref, q_ref, a0, a1, wa, bl, bg, ws, bs, o_ref):
        hw1 = jnp.dot(h1_ref[...], wa[...], preferred_element_type=jnp.float32)
        h2 = hw1 + q_ref[0] + q_ref[1] + bg[...]
        hid = jnp.dot(x_ref[...], a0[...], preferred_element_type=jnp.float32)
        hid = hid + jnp.dot(h2, a1[...], preferred_element_type=jnp.float32)
        hid = jnp.maximum(hid + bl[...], 0.0)
        lg = jnp.dot(hid, ws[...], preferred_element_type=jnp.float32) + bs[...]
        m = jnp.max(lg, axis=1, keepdims=True)
        ls = jnp.log(jnp.sum(jnp.exp(lg - m), axis=1, keepdims=True)) + m
        o_ref[...] = lg - ls

    return pl.pallas_call(
        body,
        grid=(n // bn,),
        in_specs=[
            pl.BlockSpec((bn, d), lambda i: (i, 0)),
            pl.BlockSpec((bn, h), lambda i: (i, 0)),
            pl.BlockSpec((2, bn, h), lambda i: (0, i, 0)),
            pl.BlockSpec((d, h), lambda i: (0, 0)),
            pl.BlockSpec((h, h), lambda i: (0, 0)),
            pl.BlockSpec((h, h), lambda i: (0, 0)),
            pl.BlockSpec((1, h), lambda i: (0, 0)),
            pl.BlockSpec((1, h), lambda i: (0, 0)),
            pl.BlockSpec((h, c), lambda i: (0, 0)),
            pl.BlockSpec((1, c), lambda i: (0, 0)),
        ],
        out_specs=pl.BlockSpec((bn, c), lambda i: (i, 0)),
        out_shape=jax.ShapeDtypeStruct((n, c), jnp.float32),
    )(x, h1, q, wl0, wl1, w1, b_lin, b_gc, w_smax, b_smax)


def kernel(x, edge_index, edge_norm, edge_type, W_rel, W_root, b_rgcn,
           W1, W2, b_gc, W_lin, b_lin, W_smax, b_smax):
    n, d = x.shape
    e = edge_index.shape[1]
    r, _, h = W_rel.shape

    src = edge_index[0].astype(jnp.int32)
    dst = edge_index[1].astype(jnp.int32)
    etype = edge_type.astype(jnp.int32)

    xr = _tc_rel(x, W_rel).reshape(r * n, h)
    gidx = _tc_gidx(src, etype, n)
    p1 = _edge_aggregate(xr, gidx, dst, edge_norm, n, scale=True)
    h1, hw2 = _tc_mid(p1, x, W_root, W2, b_rgcn.reshape(1, h))

    p2 = _edge_aggregate(hw2, src, dst, edge_norm, n, scale=False)

    return _tc_head(x, h1, p2, W_lin[:d], W_lin[d:], W1, b_lin.reshape(1, h),
                    b_gc.reshape(1, h), W_smax, b_smax.reshape(1, -1))


# speedup vs baseline: 1.0292x; 1.0000x over previous
import functools

import jax
import jax.numpy as jnp
from jax import lax
from jax.experimental import pallas as pl
from jax.experimental.pallas import tpu as pltpu
from jax.experimental.pallas import tpu_sc as plsc

def _bcast_lane(vec, lane):
    idx = jnp.full((LANES,), lane, jnp.int32)
    return lax.gather(
        vec, idx[:, None],
        lax.GatherDimensionNumbers(
            offset_dims=(), collapsed_slice_dims=(0,), start_index_map=(0,)),
        (1,), mode=lax.GatherScatterMode.PROMISE_IN_BOUNDS)


NC = 2
NS = 16
LANES = 16
CH = 80


def _edge_aggregate(table, gidx, dst, norm, n_nodes, *, scale):
    t_rows, hdim = table.shape
    e_total = gidx.shape[0]
    nw = NC * NS
    ept = e_total // nw
    nchunk = ept // CH
    wpt = 640
    last = n_nodes - (NS - 1) * wpt
    zr = 16
    groups = hdim // LANES
    assert 0 < last <= wpt and last % zr == 0 and wpt % zr == 0

    nbuf = 3
    assert nchunk % nbuf == 2 and nchunk >= 8

    mesh = plsc.VectorSubcoreMesh(core_axis_name="c", subcore_axis_name="s")

    cha0 = 48
    scratch = [
        pltpu.VMEM((ept,), jnp.int32),
        pltpu.VMEM((nbuf, cha0 if scale else CH), jnp.int32),
        pltpu.VMEM((nbuf, CH, hdim), jnp.float32),
        pltpu.VMEM((zr, hdim), jnp.float32),
        pltpu.VMEM_SHARED((n_nodes, hdim), jnp.float32),
        pltpu.SemaphoreType.DMA,
        pltpu.SemaphoreType.DMA,
        pltpu.SemaphoreType.DMA,
        pltpu.SemaphoreType.DMA,
        pltpu.SemaphoreType.DMA,
        pltpu.SemaphoreType.DMA,
        pltpu.SemaphoreType.DMA,
        pltpu.SemaphoreType.DMA,
        pltpu.SemaphoreType.DMA,
        pltpu.SemaphoreType.DMA,
    ]
    cha = cha0
    chb = CH - cha
    if scale:
        scratch += [
            pltpu.VMEM((nbuf, CH), jnp.float32),
            pltpu.VMEM((nbuf, chb), jnp.int32),
        ]

    def body(table_h, gidx_h, dst_h, norm_h, out_h, idx_v, dstrow_v, rows_v,
             zero_v, agg, sem_i, sem_g0, sem_g1, sem_g2, sem_d0, sem_d1,
             sem_d2, sem_s0, sem_s1, sem_s2, *opt):
        cid = lax.axis_index("c")
        sid = lax.axis_index("s")
        wid = cid * NS + sid
        ebase = pl.multiple_of(wid * ept, 8)
        sem_g = (sem_g0, sem_g1, sem_g2)
        sem_d = (sem_d0, sem_d1, sem_d2)
        sem_s = (sem_s0, sem_s1, sem_s2)
        normrow_v = opt[0] if scale else None
        dstrowb_v = opt[1] if scale else None

        pltpu.async_copy(gidx_h.at[pl.ds(ebase, ept)], idx_v, sem_i)

        def zfill(i, _):
            row = i // groups
            g = i % groups
            zero_v[row, pl.ds(g * LANES, LANES)] = jnp.zeros((LANES,), jnp.float32)
            return 0
        lax.fori_loop(0, zr * groups, zfill, 0)
        nbase = pl.multiple_of(sid * wpt, 8)

        @pl.when(sid < NS - 1)
        def _zero_full():
            for k in range(wpt // zr):
                pltpu.async_copy(zero_v, agg.at[pl.ds(nbase + k * zr, zr)], sem_s0)
            for k in range(wpt // zr):
                pltpu.make_async_copy(
                    zero_v, agg.at[pl.ds(nbase + k * zr, zr)], sem_s0).wait()

        @pl.when(sid == NS - 1)
        def _zero_last():
            for k in range(last // zr):
                pltpu.async_copy(zero_v, agg.at[pl.ds(nbase + k * zr, zr)], sem_s0)
            for k in range(last // zr):
                pltpu.make_async_copy(
                    zero_v, agg.at[pl.ds(nbase + k * zr, zr)], sem_s0).wait()

        pltpu.make_async_copy(gidx_h.at[pl.ds(ebase, ept)], idx_v, sem_i).wait()

        plsc.subcore_barrier()

        def g_off(j):
            return pl.multiple_of(j * CH, 8)

        def issue_fetch(j, bb):
            off = g_off(j)
            pltpu.async_copy(table_h.at[idx_v.at[pl.ds(off, CH)]],
                             rows_v.at[bb], sem_g[bb])
            if scale:
                pltpu.async_copy(dst_h.at[pl.ds(ebase + off, cha)],
                                 dstrow_v.at[bb], sem_d[bb])
                pltpu.async_copy(dst_h.at[pl.ds(ebase + off + cha, chb)],
                                 dstrowb_v.at[bb], sem_d[bb])
                pltpu.async_copy(norm_h.at[pl.ds(ebase + off, CH)],
                                 normrow_v.at[bb], sem_d[bb])
            else:
                pltpu.async_copy(dst_h.at[pl.ds(ebase + off, CH)],
                                 dstrow_v.at[bb], sem_d[bb])

        def wait_fetch(j, bb):
            off = g_off(j)
            pltpu.make_async_copy(table_h.at[idx_v.at[pl.ds(off, CH)]],
                                  rows_v.at[bb], sem_g[bb]).wait()
            if scale:
                pltpu.make_async_copy(dst_h.at[pl.ds(ebase + off, cha)],
                                      dstrow_v.at[bb], sem_d[bb]).wait()
                pltpu.make_async_copy(dst_h.at[pl.ds(ebase + off + cha, chb)],
                                      dstrowb_v.at[bb], sem_d[bb]).wait()
                pltpu.make_async_copy(norm_h.at[pl.ds(ebase + off, CH)],
                                      normrow_v.at[bb], sem_d[bb]).wait()
            else:
                pltpu.make_async_copy(dst_h.at[pl.ds(ebase + off, CH)],
                                      dstrow_v.at[bb], sem_d[bb]).wait()

        def issue_scatter_a(bb):
            if scale:
                pltpu.async_copy(rows_v.at[bb, pl.ds(0, cha)],
                                 agg.at[dstrow_v.at[bb]], sem_s[bb], add=True)
            else:
                pltpu.async_copy(rows_v.at[bb], agg.at[dstrow_v.at[bb]],
                                 sem_s[bb], add=True)

        def issue_scatter_b(bb):
            if scale:
                pltpu.async_copy(rows_v.at[bb, pl.ds(cha, chb)],
                                 agg.at[dstrowb_v.at[bb]], sem_s[bb], add=True)

        def wait_scatter(bb):
            if scale:
                pltpu.make_async_copy(rows_v.at[bb, pl.ds(0, cha)],
                                      agg.at[dstrow_v.at[bb]], sem_s[bb]).wait()
                pltpu.make_async_copy(rows_v.at[bb, pl.ds(cha, chb)],
                                      agg.at[dstrowb_v.at[bb]], sem_s[bb]).wait()
            else:
                pltpu.make_async_copy(rows_v.at[bb], agg.at[dstrow_v.at[bb]],
                                      sem_s[bb]).wait()

        def do_scale(bb, glo, ghi):
            if not scale:
                return
            for g16 in range(glo, ghi):
                norm16 = normrow_v[bb, pl.ds(g16 * LANES, LANES)]

                def scale_one(i, _c, g16=g16, norm16=norm16):
                    nb = _bcast_lane(norm16, i)
                    row = g16 * LANES + i
                    for g in range(groups):
                        sl = pl.ds(g * LANES, LANES)
                        rows_v[bb, row, sl] = rows_v[bb, row, sl] * nb
                    return 0
                lax.fori_loop(0, LANES, scale_one, 0)

        def run_chunk(j, q, fetch_next, wait_prev):
            wait_fetch(j, q)
            do_scale(q, 0, cha // LANES)
            issue_scatter_a(q)
            do_scale(q, cha // LANES, CH // LANES)
            issue_scatter_b(q)
            if fetch_next:
                q2 = (q + 2) % nbuf
                if wait_prev:
                    wait_scatter(q2)
                issue_fetch(j + 2, q2)

        issue_fetch(0, 0)
        issue_fetch(1, 1)
        run_chunk(0, 0, True, False)
        run_chunk(1, 1, True, True)
        run_chunk(2, 2, True, True)

        def steady(j3, _):
            j = 3 * j3
            run_chunk(j, 0, True, True)
            run_chunk(j + 1, 1, True, True)
            run_chunk(j + 2, 2, True, True)
            return 0
        lax.fori_loop(1, 1 + (nchunk - 5) // 3, steady, 0)

        run_chunk(nchunk - 2, 0, False, False)
        run_chunk(nchunk - 1, 1, False, False)
        wait_scatter(2)
        wait_scatter(0)
        wait_scatter(1)

        plsc.subcore_barrier()

        @pl.when(sid < NS - 1)
        def _wb_full():
            pltpu.sync_copy(agg.at[pl.ds(nbase, wpt)],
                            out_h.at[cid, pl.ds(nbase, wpt)])

        @pl.when(sid == NS - 1)
        def _wb_last():
            pltpu.sync_copy(agg.at[pl.ds(nbase, last)],
                            out_h.at[cid, pl.ds(nbase, last)])

    f = pl.kernel(
        body,
        out_type=jax.ShapeDtypeStruct((NC, n_nodes, hdim), jnp.float32),
        mesh=mesh,
        scratch_types=scratch,
    )
    return f(table, gidx, dst, norm)


def _tc_rel(x, w_rel):
    r, d, h = w_rel.shape
    n = x.shape[0]
    bn = 1000

    def body(x_ref, w_ref, o_ref):
        for ri in range(r):
            o_ref[ri] = jnp.dot(x_ref[...], w_ref[ri],
                                preferred_element_type=jnp.float32)

    return pl.pallas_call(
        body,
        grid=(n // bn,),
        in_specs=[
            pl.BlockSpec((bn, d), lambda i: (i, 0)),
            pl.BlockSpec((r, d, h), lambda i: (0, 0, 0)),
        ],
        out_specs=pl.BlockSpec((r, bn, h), lambda i: (0, i, 0)),
        out_shape=jax.ShapeDtypeStruct((r, n, h), jnp.float32),
    )(x, w_rel)


def _tc_gidx(src, etype, n_nodes):
    e = src.shape[0]

    def body(s_ref, t_ref, g_ref):
        g_ref[...] = t_ref[...] * n_nodes + s_ref[...]

    return pl.pallas_call(
        body,
        out_shape=jax.ShapeDtypeStruct((e,), jnp.int32),
    )(src, etype)


def _tc_mid(p, x, w_root, w2, b_rgcn):
    n, d = x.shape
    h = w_root.shape[1]
    bn = 1000

    def body(p_ref, x_ref, wr, wb, b_ref, o1, o2):
        h1 = (p_ref[0] + p_ref[1] + b_ref[...]
              + jnp.dot(x_ref[...], wr[...], preferred_element_type=jnp.float32))
        o1[...] = h1
        o2[...] = jnp.dot(h1, wb[...], preferred_element_type=jnp.float32)

    return pl.pallas_call(
        body,
        grid=(n // bn,),
        in_specs=[
            pl.BlockSpec((2, bn, h), lambda i: (0, i, 0)),
            pl.BlockSpec((bn, d), lambda i: (i, 0)),
            pl.BlockSpec((d, h), lambda i: (0, 0)),
            pl.BlockSpec((h, h), lambda i: (0, 0)),
            pl.BlockSpec((1, h), lambda i: (0, 0)),
        ],
        out_specs=[
            pl.BlockSpec((bn, h), lambda i: (i, 0)),
            pl.BlockSpec((bn, h), lambda i: (i, 0)),
        ],
        out_shape=[
            jax.ShapeDtypeStruct((n, h), jnp.float32),
            jax.ShapeDtypeStruct((n, h), jnp.float32),
        ],
    )(p, x, w_root, w2, b_rgcn)


def _tc_head(x, h1, q, wl0, wl1, w1, b_lin, b_gc, w_smax, b_smax):
    n, d = x.shape
    h = w1.shape[1]
    c = w_smax.shape[1]
    bn = 1000

    def body(x_ref, h1_ref, q_ref, a0, a1, wa, bl, bg, ws, bs, o_ref):
        hw1 = jnp.dot(h1_ref[...], wa[...], preferred_element_type=jnp.float32)
        h2 = hw1 + q_ref[0] + q_ref[1] + bg[...]
        hid = jnp.dot(x_ref[...], a0[...], preferred_element_type=jnp.float32)
        hid = hid + jnp.dot(h2, a1[...], preferred_element_type=jnp.float32)
        hid = jnp.maximum(hid + bl[...], 0.0)
        lg = jnp.dot(hid, ws[...], preferred_element_type=jnp.float32) + bs[...]
        m = jnp.max(lg, axis=1, keepdims=True)
        ls = jnp.log(jnp.sum(jnp.exp(lg - m), axis=1, keepdims=True)) + m
        o_ref[...] = lg - ls

    return pl.pallas_call(
        body,
        grid=(n // bn,),
        in_specs=[
            pl.BlockSpec((bn, d), lambda i: (i, 0)),
            pl.BlockSpec((bn, h), lambda i: (i, 0)),
            pl.BlockSpec((2, bn, h), lambda i: (0, i, 0)),
            pl.BlockSpec((d, h), lambda i: (0, 0)),
            pl.BlockSpec((h, h), lambda i: (0, 0)),
            pl.BlockSpec((h, h), lambda i: (0, 0)),
            pl.BlockSpec((1, h), lambda i: (0, 0)),
            pl.BlockSpec((1, h), lambda i: (0, 0)),
            pl.BlockSpec((h, c), lambda i: (0, 0)),
            pl.BlockSpec((1, c), lambda i: (0, 0)),
        ],
        out_specs=pl.BlockSpec((bn, c), lambda i: (i, 0)),
        out_shape=jax.ShapeDtypeStruct((n, c), jnp.float32),
    )(x, h1, q, wl0, wl1, w1, b_lin, b_gc, w_smax, b_smax)


def kernel(x, edge_index, edge_norm, edge_type, W_rel, W_root, b_rgcn,
           W1, W2, b_gc, W_lin, b_lin, W_smax, b_smax):
    n, d = x.shape
    e = edge_index.shape[1]
    r, _, h = W_rel.shape

    src = edge_index[0].astype(jnp.int32)
    dst = edge_index[1].astype(jnp.int32)
    etype = edge_type.astype(jnp.int32)

    xr = _tc_rel(x, W_rel).reshape(r * n, h)
    gidx = _tc_gidx(src, etype, n)
    p1 = _edge_aggregate(xr, gidx, dst, edge_norm, n, scale=True)
    h1, hw2 = _tc_mid(p1, x, W_root, W2, b_rgcn.reshape(1, h))

    p2 = _edge_aggregate(hw2, src, dst, edge_norm, n, scale=False)

    return _tc_head(x, h1, p2, W_lin[:d], W_lin[d:], W1, b_lin.reshape(1, h),
                    b_gc.reshape(1, h), W_smax, b_smax.reshape(1, -1))
